# baseline (speedup 1.0000x reference)
import functools

import jax
import jax.numpy as jnp
from jax import lax
from jax.experimental import pallas as pl
from jax.experimental.pallas import tpu as pltpu
from jax.experimental.pallas import tpu_sc as plsc

K = 64
D = 4096
SPLIT = 4
DC = D // SPLIT
NROWS = K * SPLIT
RPW = 8

_mesh = plsc.VectorSubcoreMesh(core_axis_name="c", subcore_axis_name="s")


@functools.partial(
    pl.kernel,
    mesh=_mesh,
    out_type=jax.ShapeDtypeStruct((K, SPLIT, DC), jnp.float32),
    scratch_types=[
        pltpu.VMEM((80,), jnp.int32),
        pltpu.VMEM((16,), jnp.int32),
        pltpu.VMEM((RPW, DC), jnp.float32),
        pltpu.SemaphoreType.DMA,
    ],
)
def _gather(table_hbm, ids_hbm, out_hbm, ids_v, fidx_v, rows_v, sem):
    info = plsc.get_sparse_core_info()
    wid = lax.axis_index("s") * info.num_cores + lax.axis_index("c")
    rbase = (wid % 8) * RPW
    chunk = wid // 8

    pltpu.sync_copy(ids_hbm, ids_v.at[pl.ds(0, K)])
    vec = ids_v[pl.ds(rbase, 16)]
    fidx = vec * SPLIT + chunk
    fidx_v[...] = jnp.clip(fidx, 0, NROWS - 1)
    pltpu.async_copy(table_hbm.at[fidx_v.at[pl.ds(0, RPW)]], rows_v, sem).wait()
    pltpu.sync_copy(rows_v, out_hbm.at[pl.ds(rbase, RPW), chunk])


def kernel(table, position_ids):
    out = _gather(table.reshape(NROWS, DC), position_ids.astype(jnp.int32))
    return out.reshape(K, D)
